# baseline (speedup 1.0000x reference)
import functools

import jax
import jax.numpy as jnp
from jax import lax
from jax.experimental import pallas as pl
from jax.experimental.pallas import tpu as pltpu
from jax.experimental.pallas import tpu_sc as plsc

VOCAB = 1000000
LENGTH = 200
DIM = 64
BATCH = 4096

B = BATCH * LENGTH
NC, NS = 2, 16
NW = NC * NS
SEQW = BATCH // NW
BPW = B // NW
STEP = 100
SEQ_PER_CHUNK = 2
CHUNK = SEQ_PER_CHUNK * LENGTH
CHUNK_STEPS = CHUNK // STEP
NCHUNK = BPW // CHUNK
NSTEPS = BPW // STEP
NBUF = 2

LPAD = LENGTH // 2 + 4
OUTROWS = BATCH * 2 * LPAD

CBLK = 512
CGRID = (VOCAB + CBLK - 1) // CBLK


def _conv_body(tt_ref, out_ref, scr_ref):
    scr_ref[...] = tt_ref[...].T
    out_ref[...] = jnp.concatenate(
        [scr_ref[::2, :], scr_ref[1::2, :]], axis=1
    )


_convert = pl.pallas_call(
    _conv_body,
    grid=(CGRID,),
    in_specs=[pl.BlockSpec((DIM, CBLK), lambda i: (0, i))],
    out_specs=pl.BlockSpec((CBLK // 2, 2 * DIM), lambda i: (i, 0)),
    out_shape=jax.ShapeDtypeStruct((VOCAB // 2, 2 * DIM), jnp.float32),
    scratch_shapes=[pltpu.VMEM((CBLK, DIM), jnp.float32)],
)

_mesh = plsc.VectorSubcoreMesh(core_axis_name="c", subcore_axis_name="s")


@functools.partial(
    pl.kernel,
    out_type=jax.ShapeDtypeStruct((OUTROWS, DIM), jnp.float32),
    mesh=_mesh,
    scratch_types=[
        pltpu.VMEM((NSTEPS, STEP), jnp.int32),
        pltpu.VMEM((NBUF, CHUNK, DIM), jnp.float32),
        pltpu.SemaphoreType.DMA((NBUF,)),
    ],
    compiler_params=pltpu.CompilerParams(use_tc_tiling_on_sc=False),
)
def _embed(x_hbm, tok_hbm, out_hbm, idx_v, dest_v, sems):
    wid = lax.axis_index("s") * NC + lax.axis_index("c")

    pltpu.sync_copy(x_hbm.at[pl.ds(wid * NSTEPS, NSTEPS)], idx_v)

    def fire(c, b):
        for s in range(CHUNK_STEPS):
            pltpu.async_copy(
                tok_hbm.at[idx_v.at[c * CHUNK_STEPS + s]],
                dest_v.at[b, pl.ds(s * STEP, STEP)],
                sems.at[b],
            )

    def drain(b):
        pltpu.make_async_copy(
            tok_hbm.at[pl.ds(0, CHUNK)], dest_v.at[b], sems.at[b]
        ).wait()

    for b in range(NBUF):
        fire(b, b)

    @pl.loop(0, NCHUNK, step=NBUF)
    def _chunks(c):
        for b in range(NBUF):
            cc = c + b
            drain(b)
            seq0 = wid * SEQW + cc * SEQ_PER_CHUNK
            for q in range(SEQ_PER_CHUNK):
                pltpu.sync_copy(
                    dest_v.at[b, pl.ds(q * LENGTH, LENGTH)],
                    out_hbm.at[pl.ds((seq0 + q) * 2 * LPAD, LENGTH)],
                )

            @pl.when(cc + NBUF < NCHUNK)
            def _refill():
                fire(cc + NBUF, b)


PB = 512
PL = 8
PGRID = (BATCH // PB, LPAD // PL)


def _post_body(g_ref, pos_ref, out_ref):
    for k in range(PL):
        t = g_ref[:, k, :].T
        out_ref[2 * k, :, :] = t[0:DIM, :] + pos_ref[2 * k, :][:, None]
        out_ref[2 * k + 1, :, :] = t[DIM:, :] + pos_ref[2 * k + 1, :][:, None]


_post = pl.pallas_call(
    _post_body,
    grid=PGRID,
    in_specs=[
        pl.BlockSpec((PB, PL, 2 * DIM), lambda bb, lq: (bb, lq, 0)),
        pl.BlockSpec((2 * PL, DIM), lambda bb, lq: (lq, 0)),
    ],
    out_specs=pl.BlockSpec((2 * PL, DIM, PB), lambda bb, lq: (lq, 0, bb)),
    out_shape=jax.ShapeDtypeStruct((LENGTH, DIM, BATCH), jnp.float32),
)


def kernel(x, token_table, pos_table):
    xi = x.reshape(B).astype(jnp.int32).reshape(B // STEP, STEP)
    tab_lin = _convert(token_table.T).reshape(VOCAB, DIM)
    gathered = _embed(xi, tab_lin)
    g3 = gathered.reshape(BATCH, LPAD, 2 * DIM)
    out = _post(g3, pos_table)
    return jnp.transpose(out, (2, 0, 1))


# speedup vs baseline: 1.9437x; 1.9437x over previous
import functools

import jax
import jax.numpy as jnp
from jax import lax
from jax.experimental import pallas as pl
from jax.experimental.pallas import tpu as pltpu
from jax.experimental.pallas import tpu_sc as plsc

VOCAB = 1000000
LENGTH = 200
DIM = 64
BATCH = 4096

B = BATCH * LENGTH
NC, NS = 2, 16
NW = NC * NS
SEQW = BATCH // NW
BPW = B // NW
STEP = 100
SEQ_PER_CHUNK = 2
CHUNK = SEQ_PER_CHUNK * LENGTH
CHUNK_STEPS = CHUNK // STEP
NCHUNK = BPW // CHUNK
NSTEPS = BPW // STEP
NBUF = 2

LPAD = LENGTH // 2 + 4
OUTROWS = BATCH * 2 * LPAD

CBLK = 2048
CGRID = (VOCAB + CBLK - 1) // CBLK


def _conv_body(tt_ref, out_ref, scr_ref):
    scr_ref[...] = tt_ref[...].T
    out_ref[...] = jnp.concatenate(
        [scr_ref[::2, :], scr_ref[1::2, :]], axis=1
    )


_convert = pl.pallas_call(
    _conv_body,
    grid=(CGRID,),
    in_specs=[pl.BlockSpec((DIM, CBLK), lambda i: (0, i))],
    out_specs=pl.BlockSpec((CBLK // 2, 2 * DIM), lambda i: (i, 0)),
    out_shape=jax.ShapeDtypeStruct((VOCAB // 2, 2 * DIM), jnp.float32),
    scratch_shapes=[pltpu.VMEM((CBLK, DIM), jnp.float32)],
)

_mesh = plsc.VectorSubcoreMesh(core_axis_name="c", subcore_axis_name="s")


@functools.partial(
    pl.kernel,
    out_type=jax.ShapeDtypeStruct((OUTROWS, DIM), jnp.float32),
    mesh=_mesh,
    scratch_types=[
        pltpu.VMEM((NSTEPS, STEP), jnp.int32),
        pltpu.VMEM((NBUF, CHUNK, DIM), jnp.float32),
        pltpu.SemaphoreType.DMA((NBUF,)),
    ],
    compiler_params=pltpu.CompilerParams(use_tc_tiling_on_sc=False),
)
def _embed(x_hbm, tok_hbm, out_hbm, idx_v, dest_v, sems):
    wid = lax.axis_index("s") * NC + lax.axis_index("c")

    pltpu.sync_copy(x_hbm.at[pl.ds(wid * NSTEPS, NSTEPS)], idx_v)

    def fire(c, b):
        for s in range(CHUNK_STEPS):
            pltpu.async_copy(
                tok_hbm.at[idx_v.at[c * CHUNK_STEPS + s]],
                dest_v.at[b, pl.ds(s * STEP, STEP)],
                sems.at[b],
            )

    def drain(b):
        pltpu.make_async_copy(
            tok_hbm.at[pl.ds(0, CHUNK)], dest_v.at[b], sems.at[b]
        ).wait()

    for b in range(NBUF):
        fire(b, b)

    @pl.loop(0, NCHUNK, step=NBUF)
    def _chunks(c):
        for b in range(NBUF):
            cc = c + b
            drain(b)
            seq0 = wid * SEQW + cc * SEQ_PER_CHUNK
            for q in range(SEQ_PER_CHUNK):
                pltpu.sync_copy(
                    dest_v.at[b, pl.ds(q * LENGTH, LENGTH)],
                    out_hbm.at[pl.ds((seq0 + q) * 2 * LPAD, LENGTH)],
                )

            @pl.when(cc + NBUF < NCHUNK)
            def _refill():
                fire(cc + NBUF, b)


PB = 512
PL = 8
PGRID = (BATCH // PB, LPAD // PL)


def _post_body(g_ref, pos_ref, out_ref):
    for k in range(PL):
        t = g_ref[:, k, :].T
        out_ref[2 * k, :, :] = t[0:DIM, :] + pos_ref[2 * k, :][:, None]
        out_ref[2 * k + 1, :, :] = t[DIM:, :] + pos_ref[2 * k + 1, :][:, None]


_post = pl.pallas_call(
    _post_body,
    grid=PGRID,
    in_specs=[
        pl.BlockSpec((PB, PL, 2 * DIM), lambda bb, lq: (bb, lq, 0)),
        pl.BlockSpec((2 * PL, DIM), lambda bb, lq: (lq, 0)),
    ],
    out_specs=pl.BlockSpec((2 * PL, DIM, PB), lambda bb, lq: (lq, 0, bb)),
    out_shape=jax.ShapeDtypeStruct((LENGTH, DIM, BATCH), jnp.float32),
)


def kernel(x, token_table, pos_table):
    xi = x.reshape(B).astype(jnp.int32).reshape(B // STEP, STEP)
    tab_lin = _convert(token_table.T).reshape(VOCAB, DIM)
    gathered = _embed(xi, tab_lin)
    g3 = gathered.reshape(BATCH, LPAD, 2 * DIM)
    out = _post(g3, pos_table)
    return jnp.transpose(out, (2, 0, 1))


# speedup vs baseline: 2.2871x; 1.1766x over previous
import functools

import jax
import jax.numpy as jnp
from jax import lax
from jax.experimental import pallas as pl
from jax.experimental.pallas import tpu as pltpu
from jax.experimental.pallas import tpu_sc as plsc

VOCAB = 1000000
LENGTH = 200
DIM = 64
BATCH = 4096

B = BATCH * LENGTH
NC, NS = 2, 16
NW = NC * NS
SEQW = BATCH // NW
BPW = B // NW
STEP = 100
SEQ_PER_CHUNK = 2
CHUNK = SEQ_PER_CHUNK * LENGTH
CHUNK_STEPS = CHUNK // STEP
NCHUNK = BPW // CHUNK
NSTEPS = BPW // STEP
NBUF = 2

LPAD = LENGTH // 2 + 4
OUTROWS = BATCH * 2 * LPAD

CBLK = 4096
CGRID = (VOCAB + CBLK - 1) // CBLK


def _conv_body(tt_ref, out_ref, scr_ref):
    scr_ref[...] = tt_ref[...].T
    out_ref[...] = jnp.concatenate(
        [scr_ref[::2, :], scr_ref[1::2, :]], axis=1
    )


_convert = pl.pallas_call(
    _conv_body,
    grid=(CGRID,),
    in_specs=[pl.BlockSpec((DIM, CBLK), lambda i: (0, i))],
    out_specs=pl.BlockSpec((CBLK // 2, 2 * DIM), lambda i: (i, 0)),
    out_shape=jax.ShapeDtypeStruct((VOCAB // 2, 2 * DIM), jnp.float32),
    scratch_shapes=[pltpu.VMEM((CBLK, DIM), jnp.float32)],
)

_mesh = plsc.VectorSubcoreMesh(core_axis_name="c", subcore_axis_name="s")


@functools.partial(
    pl.kernel,
    out_type=jax.ShapeDtypeStruct((OUTROWS, DIM), jnp.float32),
    mesh=_mesh,
    scratch_types=[
        pltpu.VMEM((NSTEPS, STEP), jnp.int32),
        pltpu.VMEM((NBUF, CHUNK, DIM), jnp.float32),
        pltpu.SemaphoreType.DMA((NBUF,)),
    ],
    compiler_params=pltpu.CompilerParams(use_tc_tiling_on_sc=False),
)
def _embed(x_hbm, tok_hbm, out_hbm, idx_v, dest_v, sems):
    wid = lax.axis_index("s") * NC + lax.axis_index("c")

    pltpu.sync_copy(x_hbm.at[pl.ds(wid * NSTEPS, NSTEPS)], idx_v)

    def fire(c, b):
        for s in range(CHUNK_STEPS):
            pltpu.async_copy(
                tok_hbm.at[idx_v.at[c * CHUNK_STEPS + s]],
                dest_v.at[b, pl.ds(s * STEP, STEP)],
                sems.at[b],
            )

    def drain(b):
        pltpu.make_async_copy(
            tok_hbm.at[pl.ds(0, CHUNK)], dest_v.at[b], sems.at[b]
        ).wait()

    for b in range(NBUF):
        fire(b, b)

    @pl.loop(0, NCHUNK, step=NBUF)
    def _chunks(c):
        for b in range(NBUF):
            cc = c + b
            drain(b)
            seq0 = wid * SEQW + cc * SEQ_PER_CHUNK
            for q in range(SEQ_PER_CHUNK):
                pltpu.sync_copy(
                    dest_v.at[b, pl.ds(q * LENGTH, LENGTH)],
                    out_hbm.at[pl.ds((seq0 + q) * 2 * LPAD, LENGTH)],
                )

            @pl.when(cc + NBUF < NCHUNK)
            def _refill():
                fire(cc + NBUF, b)


PB = 512
PL = 8
PGRID = (BATCH // PB, LPAD // PL)


def _post_body(g_ref, pos_ref, out_ref):
    for k in range(PL):
        t = g_ref[:, k, :].T
        out_ref[2 * k, :, :] = t[0:DIM, :] + pos_ref[2 * k, :][:, None]
        out_ref[2 * k + 1, :, :] = t[DIM:, :] + pos_ref[2 * k + 1, :][:, None]


_post = pl.pallas_call(
    _post_body,
    grid=PGRID,
    in_specs=[
        pl.BlockSpec((PB, PL, 2 * DIM), lambda bb, lq: (bb, lq, 0)),
        pl.BlockSpec((2 * PL, DIM), lambda bb, lq: (lq, 0)),
    ],
    out_specs=pl.BlockSpec((2 * PL, DIM, PB), lambda bb, lq: (lq, 0, bb)),
    out_shape=jax.ShapeDtypeStruct((LENGTH, DIM, BATCH), jnp.float32),
)


def kernel(x, token_table, pos_table):
    xi = x.reshape(B).astype(jnp.int32).reshape(B // STEP, STEP)
    tab_lin = _convert(token_table.T).reshape(VOCAB, DIM)
    gathered = _embed(xi, tab_lin)
    g3 = gathered.reshape(BATCH, LPAD, 2 * DIM)
    out = _post(g3, pos_table)
    return jnp.transpose(out, (2, 0, 1))


# speedup vs baseline: 2.5354x; 1.1086x over previous
import functools

import jax
import jax.numpy as jnp
from jax import lax
from jax.experimental import pallas as pl
from jax.experimental.pallas import tpu as pltpu
from jax.experimental.pallas import tpu_sc as plsc

VOCAB = 1000000
LENGTH = 200
DIM = 64
BATCH = 4096

B = BATCH * LENGTH
NC, NS = 2, 16
NW = NC * NS
SEQW = BATCH // NW
BPW = B // NW
STEP = 100
SEQ_PER_CHUNK = 2
CHUNK = SEQ_PER_CHUNK * LENGTH
CHUNK_STEPS = CHUNK // STEP
NCHUNK = BPW // CHUNK
NSTEPS = BPW // STEP
NBUF = 2

LPAD = LENGTH // 2 + 4
OUTROWS = BATCH * 2 * LPAD

CBLK = 8192
CGRID = (VOCAB + CBLK - 1) // CBLK


def _conv_body(tt_ref, out_ref, scr_ref):
    scr_ref[...] = tt_ref[...].T
    out_ref[...] = jnp.concatenate(
        [scr_ref[::2, :], scr_ref[1::2, :]], axis=1
    )


_convert = pl.pallas_call(
    _conv_body,
    grid=(CGRID,),
    in_specs=[pl.BlockSpec((DIM, CBLK), lambda i: (0, i))],
    out_specs=pl.BlockSpec((CBLK // 2, 2 * DIM), lambda i: (i, 0)),
    out_shape=jax.ShapeDtypeStruct((VOCAB // 2, 2 * DIM), jnp.float32),
    scratch_shapes=[pltpu.VMEM((CBLK, DIM), jnp.float32)],
)

_mesh = plsc.VectorSubcoreMesh(core_axis_name="c", subcore_axis_name="s")


@functools.partial(
    pl.kernel,
    out_type=jax.ShapeDtypeStruct((OUTROWS, DIM), jnp.float32),
    mesh=_mesh,
    scratch_types=[
        pltpu.VMEM((NSTEPS, STEP), jnp.int32),
        pltpu.VMEM((NBUF, CHUNK, DIM), jnp.float32),
        pltpu.SemaphoreType.DMA((NBUF,)),
    ],
    compiler_params=pltpu.CompilerParams(use_tc_tiling_on_sc=False),
)
def _embed(x_hbm, tok_hbm, out_hbm, idx_v, dest_v, sems):
    wid = lax.axis_index("s") * NC + lax.axis_index("c")

    pltpu.sync_copy(x_hbm.at[pl.ds(wid * NSTEPS, NSTEPS)], idx_v)

    def fire(c, b):
        for s in range(CHUNK_STEPS):
            pltpu.async_copy(
                tok_hbm.at[idx_v.at[c * CHUNK_STEPS + s]],
                dest_v.at[b, pl.ds(s * STEP, STEP)],
                sems.at[b],
            )

    def drain(b):
        pltpu.make_async_copy(
            tok_hbm.at[pl.ds(0, CHUNK)], dest_v.at[b], sems.at[b]
        ).wait()

    for b in range(NBUF):
        fire(b, b)

    @pl.loop(0, NCHUNK, step=NBUF)
    def _chunks(c):
        for b in range(NBUF):
            cc = c + b
            drain(b)
            seq0 = wid * SEQW + cc * SEQ_PER_CHUNK
            for q in range(SEQ_PER_CHUNK):
                pltpu.sync_copy(
                    dest_v.at[b, pl.ds(q * LENGTH, LENGTH)],
                    out_hbm.at[pl.ds((seq0 + q) * 2 * LPAD, LENGTH)],
                )

            @pl.when(cc + NBUF < NCHUNK)
            def _refill():
                fire(cc + NBUF, b)


PB = 512
PL = 8
PGRID = (BATCH // PB, LPAD // PL)


def _post_body(g_ref, pos_ref, out_ref):
    for k in range(PL):
        t = g_ref[:, k, :].T
        out_ref[2 * k, :, :] = t[0:DIM, :] + pos_ref[2 * k, :][:, None]
        out_ref[2 * k + 1, :, :] = t[DIM:, :] + pos_ref[2 * k + 1, :][:, None]


_post = pl.pallas_call(
    _post_body,
    grid=PGRID,
    in_specs=[
        pl.BlockSpec((PB, PL, 2 * DIM), lambda bb, lq: (bb, lq, 0)),
        pl.BlockSpec((2 * PL, DIM), lambda bb, lq: (lq, 0)),
    ],
    out_specs=pl.BlockSpec((2 * PL, DIM, PB), lambda bb, lq: (lq, 0, bb)),
    out_shape=jax.ShapeDtypeStruct((LENGTH, DIM, BATCH), jnp.float32),
)


def kernel(x, token_table, pos_table):
    xi = x.reshape(B).astype(jnp.int32).reshape(B // STEP, STEP)
    tab_lin = _convert(token_table.T).reshape(VOCAB, DIM)
    gathered = _embed(xi, tab_lin)
    g3 = gathered.reshape(BATCH, LPAD, 2 * DIM)
    out = _post(g3, pos_table)
    return jnp.transpose(out, (2, 0, 1))


# speedup vs baseline: 2.6748x; 1.0550x over previous
import functools

import jax
import jax.numpy as jnp
from jax import lax
from jax.experimental import pallas as pl
from jax.experimental.pallas import tpu as pltpu
from jax.experimental.pallas import tpu_sc as plsc

VOCAB = 1000000
LENGTH = 200
DIM = 64
BATCH = 4096

B = BATCH * LENGTH
NC, NS = 2, 16
NW = NC * NS
SEQW = BATCH // NW
BPW = B // NW
STEP = 100
SEQ_PER_CHUNK = 2
CHUNK = SEQ_PER_CHUNK * LENGTH
CHUNK_STEPS = CHUNK // STEP
NCHUNK = BPW // CHUNK
NSTEPS = BPW // STEP
NBUF = 2

LPAD = LENGTH // 2 + 4
OUTROWS = BATCH * 2 * LPAD

CBLK = 16384
CGRID = (VOCAB + CBLK - 1) // CBLK


def _conv_body(tt_ref, out_ref, scr_ref):
    scr_ref[...] = tt_ref[...].T
    out_ref[...] = jnp.concatenate(
        [scr_ref[::2, :], scr_ref[1::2, :]], axis=1
    )


_convert = pl.pallas_call(
    _conv_body,
    grid=(CGRID,),
    in_specs=[pl.BlockSpec((DIM, CBLK), lambda i: (0, i))],
    out_specs=pl.BlockSpec((CBLK // 2, 2 * DIM), lambda i: (i, 0)),
    out_shape=jax.ShapeDtypeStruct((VOCAB // 2, 2 * DIM), jnp.float32),
    scratch_shapes=[pltpu.VMEM((CBLK, DIM), jnp.float32)],
)

_mesh = plsc.VectorSubcoreMesh(core_axis_name="c", subcore_axis_name="s")


@functools.partial(
    pl.kernel,
    out_type=jax.ShapeDtypeStruct((OUTROWS, DIM), jnp.float32),
    mesh=_mesh,
    scratch_types=[
        pltpu.VMEM((NSTEPS, STEP), jnp.int32),
        pltpu.VMEM((NBUF, CHUNK, DIM), jnp.float32),
        pltpu.SemaphoreType.DMA((NBUF,)),
    ],
    compiler_params=pltpu.CompilerParams(use_tc_tiling_on_sc=False),
)
def _embed(x_hbm, tok_hbm, out_hbm, idx_v, dest_v, sems):
    wid = lax.axis_index("s") * NC + lax.axis_index("c")

    pltpu.sync_copy(x_hbm.at[pl.ds(wid * NSTEPS, NSTEPS)], idx_v)

    def fire(c, b):
        for s in range(CHUNK_STEPS):
            pltpu.async_copy(
                tok_hbm.at[idx_v.at[c * CHUNK_STEPS + s]],
                dest_v.at[b, pl.ds(s * STEP, STEP)],
                sems.at[b],
            )

    def drain(b):
        pltpu.make_async_copy(
            tok_hbm.at[pl.ds(0, CHUNK)], dest_v.at[b], sems.at[b]
        ).wait()

    for b in range(NBUF):
        fire(b, b)

    @pl.loop(0, NCHUNK, step=NBUF)
    def _chunks(c):
        for b in range(NBUF):
            cc = c + b
            drain(b)
            seq0 = wid * SEQW + cc * SEQ_PER_CHUNK
            for q in range(SEQ_PER_CHUNK):
                pltpu.sync_copy(
                    dest_v.at[b, pl.ds(q * LENGTH, LENGTH)],
                    out_hbm.at[pl.ds((seq0 + q) * 2 * LPAD, LENGTH)],
                )

            @pl.when(cc + NBUF < NCHUNK)
            def _refill():
                fire(cc + NBUF, b)


PB = 512
PL = 8
PGRID = (BATCH // PB, LPAD // PL)


def _post_body(g_ref, pos_ref, out_ref):
    for k in range(PL):
        t = g_ref[:, k, :].T
        out_ref[2 * k, :, :] = t[0:DIM, :] + pos_ref[2 * k, :][:, None]
        out_ref[2 * k + 1, :, :] = t[DIM:, :] + pos_ref[2 * k + 1, :][:, None]


_post = pl.pallas_call(
    _post_body,
    grid=PGRID,
    in_specs=[
        pl.BlockSpec((PB, PL, 2 * DIM), lambda bb, lq: (bb, lq, 0)),
        pl.BlockSpec((2 * PL, DIM), lambda bb, lq: (lq, 0)),
    ],
    out_specs=pl.BlockSpec((2 * PL, DIM, PB), lambda bb, lq: (lq, 0, bb)),
    out_shape=jax.ShapeDtypeStruct((LENGTH, DIM, BATCH), jnp.float32),
)


def kernel(x, token_table, pos_table):
    xi = x.reshape(B).astype(jnp.int32).reshape(B // STEP, STEP)
    tab_lin = _convert(token_table.T).reshape(VOCAB, DIM)
    gathered = _embed(xi, tab_lin)
    g3 = gathered.reshape(BATCH, LPAD, 2 * DIM)
    out = _post(g3, pos_table)
    return jnp.transpose(out, (2, 0, 1))


# speedup vs baseline: 2.7405x; 1.0246x over previous
import functools

import jax
import jax.numpy as jnp
from jax import lax
from jax.experimental import pallas as pl
from jax.experimental.pallas import tpu as pltpu
from jax.experimental.pallas import tpu_sc as plsc

VOCAB = 1000000
LENGTH = 200
DIM = 64
BATCH = 4096

B = BATCH * LENGTH
NC, NS = 2, 16
NW = NC * NS
SEQW = BATCH // NW
BPW = B // NW
STEP = 100
SEQ_PER_CHUNK = 2
CHUNK = SEQ_PER_CHUNK * LENGTH
CHUNK_STEPS = CHUNK // STEP
NCHUNK = BPW // CHUNK
NSTEPS = BPW // STEP
NBUF = 2

LPAD = LENGTH // 2 + 4
OUTROWS = BATCH * 2 * LPAD

CBLK = 32768
CGRID = (VOCAB + CBLK - 1) // CBLK


def _conv_body(tt_ref, out_ref, scr_ref):
    scr_ref[...] = tt_ref[...].T
    out_ref[...] = jnp.concatenate(
        [scr_ref[::2, :], scr_ref[1::2, :]], axis=1
    )


_convert = pl.pallas_call(
    _conv_body,
    grid=(CGRID,),
    in_specs=[pl.BlockSpec((DIM, CBLK), lambda i: (0, i))],
    out_specs=pl.BlockSpec((CBLK // 2, 2 * DIM), lambda i: (i, 0)),
    out_shape=jax.ShapeDtypeStruct((VOCAB // 2, 2 * DIM), jnp.float32),
    scratch_shapes=[pltpu.VMEM((CBLK, DIM), jnp.float32)],
)

_mesh = plsc.VectorSubcoreMesh(core_axis_name="c", subcore_axis_name="s")


@functools.partial(
    pl.kernel,
    out_type=jax.ShapeDtypeStruct((OUTROWS, DIM), jnp.float32),
    mesh=_mesh,
    scratch_types=[
        pltpu.VMEM((NSTEPS, STEP), jnp.int32),
        pltpu.VMEM((NBUF, CHUNK, DIM), jnp.float32),
        pltpu.SemaphoreType.DMA((NBUF,)),
    ],
    compiler_params=pltpu.CompilerParams(use_tc_tiling_on_sc=False),
)
def _embed(x_hbm, tok_hbm, out_hbm, idx_v, dest_v, sems):
    wid = lax.axis_index("s") * NC + lax.axis_index("c")

    pltpu.sync_copy(x_hbm.at[pl.ds(wid * NSTEPS, NSTEPS)], idx_v)

    def fire(c, b):
        for s in range(CHUNK_STEPS):
            pltpu.async_copy(
                tok_hbm.at[idx_v.at[c * CHUNK_STEPS + s]],
                dest_v.at[b, pl.ds(s * STEP, STEP)],
                sems.at[b],
            )

    def drain(b):
        pltpu.make_async_copy(
            tok_hbm.at[pl.ds(0, CHUNK)], dest_v.at[b], sems.at[b]
        ).wait()

    for b in range(NBUF):
        fire(b, b)

    @pl.loop(0, NCHUNK, step=NBUF)
    def _chunks(c):
        for b in range(NBUF):
            cc = c + b
            drain(b)
            seq0 = wid * SEQW + cc * SEQ_PER_CHUNK
            for q in range(SEQ_PER_CHUNK):
                pltpu.sync_copy(
                    dest_v.at[b, pl.ds(q * LENGTH, LENGTH)],
                    out_hbm.at[pl.ds((seq0 + q) * 2 * LPAD, LENGTH)],
                )

            @pl.when(cc + NBUF < NCHUNK)
            def _refill():
                fire(cc + NBUF, b)


PB = 512
PL = 8
PGRID = (BATCH // PB, LPAD // PL)


def _post_body(g_ref, pos_ref, out_ref):
    for k in range(PL):
        t = g_ref[:, k, :].T
        out_ref[2 * k, :, :] = t[0:DIM, :] + pos_ref[2 * k, :][:, None]
        out_ref[2 * k + 1, :, :] = t[DIM:, :] + pos_ref[2 * k + 1, :][:, None]


_post = pl.pallas_call(
    _post_body,
    grid=PGRID,
    in_specs=[
        pl.BlockSpec((PB, PL, 2 * DIM), lambda bb, lq: (bb, lq, 0)),
        pl.BlockSpec((2 * PL, DIM), lambda bb, lq: (lq, 0)),
    ],
    out_specs=pl.BlockSpec((2 * PL, DIM, PB), lambda bb, lq: (lq, 0, bb)),
    out_shape=jax.ShapeDtypeStruct((LENGTH, DIM, BATCH), jnp.float32),
)


def kernel(x, token_table, pos_table):
    xi = x.reshape(B).astype(jnp.int32).reshape(B // STEP, STEP)
    tab_lin = _convert(token_table.T).reshape(VOCAB, DIM)
    gathered = _embed(xi, tab_lin)
    g3 = gathered.reshape(BATCH, LPAD, 2 * DIM)
    out = _post(g3, pos_table)
    return jnp.transpose(out, (2, 0, 1))


# speedup vs baseline: 2.7892x; 1.0178x over previous
import functools

import jax
import jax.numpy as jnp
from jax import lax
from jax.experimental import pallas as pl
from jax.experimental.pallas import tpu as pltpu
from jax.experimental.pallas import tpu_sc as plsc

VOCAB = 1000000
LENGTH = 200
DIM = 64
BATCH = 4096

B = BATCH * LENGTH
NC, NS = 2, 16
NW = NC * NS
SEQW = BATCH // NW
BPW = B // NW
STEP = 100
SEQ_PER_CHUNK = 2
CHUNK = SEQ_PER_CHUNK * LENGTH
CHUNK_STEPS = CHUNK // STEP
NCHUNK = BPW // CHUNK
NSTEPS = BPW // STEP
NBUF = 2

LPAD = LENGTH // 2 + 4
OUTROWS = BATCH * 2 * LPAD

CBLK = 16384
CGRID = (VOCAB + CBLK - 1) // CBLK


def _conv_body(tt_ref, out_ref, scr_ref):
    scr_ref[...] = tt_ref[...].T
    out_ref[...] = jnp.concatenate(
        [scr_ref[::2, :], scr_ref[1::2, :]], axis=1
    )


_convert = pl.pallas_call(
    _conv_body,
    grid=(CGRID,),
    in_specs=[pl.BlockSpec((DIM, CBLK), lambda i: (0, i))],
    out_specs=pl.BlockSpec((CBLK // 2, 2 * DIM), lambda i: (i, 0)),
    out_shape=jax.ShapeDtypeStruct((VOCAB // 2, 2 * DIM), jnp.float32),
    scratch_shapes=[pltpu.VMEM((CBLK, DIM), jnp.float32)],
)

SL = 2
BATCH_H = BATCH // SL
BH = BATCH_H * LENGTH
SEQW_H = BATCH_H // NW
BPW_H = BH // NW
NCHUNK_H = BPW_H // CHUNK
NSTEPS_H = BPW_H // STEP
OUTROWS_H = BATCH_H * 2 * LPAD

_mesh = plsc.VectorSubcoreMesh(core_axis_name="c", subcore_axis_name="s")


@functools.partial(
    pl.kernel,
    out_type=jax.ShapeDtypeStruct((OUTROWS_H, DIM), jnp.float32),
    mesh=_mesh,
    scratch_types=[
        pltpu.VMEM((NSTEPS_H, STEP), jnp.int32),
        pltpu.VMEM((NBUF, CHUNK, DIM), jnp.float32),
        pltpu.SemaphoreType.DMA((NBUF,)),
    ],
    compiler_params=pltpu.CompilerParams(use_tc_tiling_on_sc=False),
)
def _embed(x_hbm, tok_hbm, out_hbm, idx_v, dest_v, sems):
    wid = lax.axis_index("s") * NC + lax.axis_index("c")

    pltpu.sync_copy(x_hbm.at[pl.ds(wid * NSTEPS_H, NSTEPS_H)], idx_v)

    def fire(c, b):
        for s in range(CHUNK_STEPS):
            pltpu.async_copy(
                tok_hbm.at[idx_v.at[c * CHUNK_STEPS + s]],
                dest_v.at[b, pl.ds(s * STEP, STEP)],
                sems.at[b],
            )

    def drain(b):
        pltpu.make_async_copy(
            tok_hbm.at[pl.ds(0, CHUNK)], dest_v.at[b], sems.at[b]
        ).wait()

    for b in range(NBUF):
        fire(b, b)

    @pl.loop(0, NCHUNK_H, step=NBUF)
    def _chunks(c):
        for b in range(NBUF):
            cc = c + b
            drain(b)
            seq0 = wid * SEQW_H + cc * SEQ_PER_CHUNK
            for q in range(SEQ_PER_CHUNK):
                pltpu.sync_copy(
                    dest_v.at[b, pl.ds(q * LENGTH, LENGTH)],
                    out_hbm.at[pl.ds((seq0 + q) * 2 * LPAD, LENGTH)],
                )

            @pl.when(cc + NBUF < NCHUNK_H)
            def _refill():
                fire(cc + NBUF, b)


PB = 512
PL = 8
PGRID = (BATCH // PB, LPAD // PL)


def _post_body0(g_ref, pos_ref, out_ref):
    for k in range(PL):
        t = g_ref[:, k, :].T
        out_ref[2 * k, :, :] = t[0:DIM, :] + pos_ref[2 * k, :][:, None]
        out_ref[2 * k + 1, :, :] = t[DIM:, :] + pos_ref[2 * k + 1, :][:, None]


def _post_body1(g_ref, pos_ref, prev_ref, out_ref):
    del prev_ref
    _post_body0(g_ref, pos_ref, out_ref)


BB_H = BATCH_H // PB

_post0 = pl.pallas_call(
    _post_body0,
    grid=(BB_H, LPAD // PL),
    in_specs=[
        pl.BlockSpec((PB, PL, 2 * DIM), lambda bb, lq: (bb, lq, 0)),
        pl.BlockSpec((2 * PL, DIM), lambda bb, lq: (lq, 0)),
    ],
    out_specs=pl.BlockSpec((2 * PL, DIM, PB), lambda bb, lq: (lq, 0, bb)),
    out_shape=jax.ShapeDtypeStruct((LENGTH, DIM, BATCH), jnp.float32),
)

_post1 = pl.pallas_call(
    _post_body1,
    grid=(BB_H, LPAD // PL),
    in_specs=[
        pl.BlockSpec((PB, PL, 2 * DIM), lambda bb, lq: (bb, lq, 0)),
        pl.BlockSpec((2 * PL, DIM), lambda bb, lq: (lq, 0)),
        pl.BlockSpec(memory_space=pl.ANY),
    ],
    out_specs=pl.BlockSpec((2 * PL, DIM, PB), lambda bb, lq: (lq, 0, bb + BB_H)),
    out_shape=jax.ShapeDtypeStruct((LENGTH, DIM, BATCH), jnp.float32),
    input_output_aliases={2: 0},
)


def kernel(x, token_table, pos_table):
    xi = x.reshape(B).astype(jnp.int32).reshape(SL, BH // STEP, STEP)
    tab_lin = _convert(token_table.T).reshape(VOCAB, DIM)
    g0 = _embed(xi[0], tab_lin).reshape(BATCH_H, LPAD, 2 * DIM)
    g1 = _embed(xi[1], tab_lin).reshape(BATCH_H, LPAD, 2 * DIM)
    out = _post0(g0, pos_table)
    out = _post1(g1, pos_table, out)
    return jnp.transpose(out, (2, 0, 1))


# speedup vs baseline: 2.8400x; 1.0182x over previous
import functools

import jax
import jax.numpy as jnp
from jax import lax
from jax.experimental import pallas as pl
from jax.experimental.pallas import tpu as pltpu
from jax.experimental.pallas import tpu_sc as plsc

VOCAB = 1000000
LENGTH = 200
DIM = 64
BATCH = 4096

B = BATCH * LENGTH
NC, NS = 2, 16
NW = NC * NS
SEQW = BATCH // NW
BPW = B // NW
STEP = 100
SEQ_PER_CHUNK = 2
CHUNK = SEQ_PER_CHUNK * LENGTH
CHUNK_STEPS = CHUNK // STEP
NCHUNK = BPW // CHUNK
NSTEPS = BPW // STEP
NBUF = 2

LPAD = LENGTH // 2 + 4
OUTROWS = BATCH * 2 * LPAD

CBLK = 16384
CGRID = (VOCAB + CBLK - 1) // CBLK


def _conv_body(tt_ref, out_ref, scr_ref):
    scr_ref[...] = tt_ref[...].T
    out_ref[...] = jnp.concatenate(
        [scr_ref[::2, :], scr_ref[1::2, :]], axis=1
    )


_convert = pl.pallas_call(
    _conv_body,
    grid=(CGRID,),
    in_specs=[pl.BlockSpec((DIM, CBLK), lambda i: (0, i))],
    out_specs=pl.BlockSpec((CBLK // 2, 2 * DIM), lambda i: (i, 0)),
    out_shape=jax.ShapeDtypeStruct((VOCAB // 2, 2 * DIM), jnp.float32),
    scratch_shapes=[pltpu.VMEM((CBLK, DIM), jnp.float32)],
)

SL = 4
BATCH_H = BATCH // SL
BH = BATCH_H * LENGTH
SEQW_H = BATCH_H // NW
BPW_H = BH // NW
NCHUNK_H = BPW_H // CHUNK
NSTEPS_H = BPW_H // STEP
OUTROWS_H = BATCH_H * 2 * LPAD

_mesh = plsc.VectorSubcoreMesh(core_axis_name="c", subcore_axis_name="s")


@functools.partial(
    pl.kernel,
    out_type=jax.ShapeDtypeStruct((OUTROWS_H, DIM), jnp.float32),
    mesh=_mesh,
    scratch_types=[
        pltpu.VMEM((NSTEPS_H, STEP), jnp.int32),
        pltpu.VMEM((NBUF, CHUNK, DIM), jnp.float32),
        pltpu.SemaphoreType.DMA((NBUF,)),
    ],
    compiler_params=pltpu.CompilerParams(use_tc_tiling_on_sc=False),
)
def _embed(x_hbm, tok_hbm, out_hbm, idx_v, dest_v, sems):
    wid = lax.axis_index("s") * NC + lax.axis_index("c")

    pltpu.sync_copy(x_hbm.at[pl.ds(wid * NSTEPS_H, NSTEPS_H)], idx_v)

    def fire(c, b):
        for s in range(CHUNK_STEPS):
            pltpu.async_copy(
                tok_hbm.at[idx_v.at[c * CHUNK_STEPS + s]],
                dest_v.at[b, pl.ds(s * STEP, STEP)],
                sems.at[b],
            )

    def drain(b):
        pltpu.make_async_copy(
            tok_hbm.at[pl.ds(0, CHUNK)], dest_v.at[b], sems.at[b]
        ).wait()

    for b in range(NBUF):
        fire(b, b)

    @pl.loop(0, NCHUNK_H, step=NBUF)
    def _chunks(c):
        for b in range(NBUF):
            cc = c + b
            drain(b)
            seq0 = wid * SEQW_H + cc * SEQ_PER_CHUNK
            for q in range(SEQ_PER_CHUNK):
                pltpu.sync_copy(
                    dest_v.at[b, pl.ds(q * LENGTH, LENGTH)],
                    out_hbm.at[pl.ds((seq0 + q) * 2 * LPAD, LENGTH)],
                )

            @pl.when(cc + NBUF < NCHUNK_H)
            def _refill():
                fire(cc + NBUF, b)


PB = 512
PL = 8
PGRID = (BATCH // PB, LPAD // PL)


def _post_body0(g_ref, pos_ref, out_ref):
    for k in range(PL):
        t = g_ref[:, k, :].T
        out_ref[2 * k, :, :] = t[0:DIM, :] + pos_ref[2 * k, :][:, None]
        out_ref[2 * k + 1, :, :] = t[DIM:, :] + pos_ref[2 * k + 1, :][:, None]


def _post_body1(g_ref, pos_ref, prev_ref, out_ref):
    del prev_ref
    _post_body0(g_ref, pos_ref, out_ref)


BB_H = BATCH_H // PB

def _make_post(s):
    in_specs = [
        pl.BlockSpec((PB, PL, 2 * DIM), lambda bb, lq: (bb, lq, 0)),
        pl.BlockSpec((2 * PL, DIM), lambda bb, lq: (lq, 0)),
    ]
    out_spec = pl.BlockSpec(
        (2 * PL, DIM, PB), lambda bb, lq, off=s * BB_H: (lq, 0, bb + off)
    )
    out_shape = jax.ShapeDtypeStruct((LENGTH, DIM, BATCH), jnp.float32)
    if s == 0:
        return pl.pallas_call(
            _post_body0, grid=(BB_H, LPAD // PL),
            in_specs=in_specs, out_specs=out_spec, out_shape=out_shape,
        )
    return pl.pallas_call(
        _post_body1, grid=(BB_H, LPAD // PL),
        in_specs=in_specs + [pl.BlockSpec(memory_space=pl.ANY)],
        out_specs=out_spec, out_shape=out_shape,
        input_output_aliases={2: 0},
    )


_posts = [_make_post(s) for s in range(SL)]


def kernel(x, token_table, pos_table):
    xi = x.reshape(B).astype(jnp.int32).reshape(SL, BH // STEP, STEP)
    tab_lin = _convert(token_table.T).reshape(VOCAB, DIM)
    gs = [
        _embed(xi[s], tab_lin).reshape(BATCH_H, LPAD, 2 * DIM)
        for s in range(SL)
    ]
    out = _posts[0](gs[0], pos_table)
    for s in range(1, SL):
        out = _posts[s](gs[s], pos_table, out)
    return jnp.transpose(out, (2, 0, 1))
